# baseline (speedup 1.0000x reference)
import functools

import jax
import jax.numpy as jnp
from jax import lax
from jax.experimental import pallas as pl
from jax.experimental.pallas import tpu as pltpu
from jax.experimental.pallas import tpu_sc as plsc

D_MODEL = 1024
LANES = 16
CHUNK = 16
NXB = 4
NRB = 2


def _pe_add_kernel(n_rows: int):
    info = plsc.get_sparse_core_info()
    num_workers = info.num_cores * info.num_subcores
    rows_per_w = n_rows // num_workers
    n_chunks = rows_per_w // CHUNK
    assert n_chunks % NXB == 0
    mesh = plsc.VectorSubcoreMesh(core_axis_name="c", subcore_axis_name="s")

    buf = lambda: pltpu.VMEM((CHUNK, D_MODEL), jnp.float32)

    @functools.partial(
        pl.kernel,
        mesh=mesh,
        out_type=jax.ShapeDtypeStruct((n_rows, D_MODEL), jnp.float32),
        scratch_types=(
            [pltpu.VMEM((rows_per_w,), jnp.int32)]
            + [buf() for _ in range(NXB + NRB)]
            + [pltpu.SemaphoreType.DMA for _ in range(NXB + NRB + NXB)]
        ),
    )
    def k(x_hbm, idx_hbm, pe_hbm, out_hbm, idx_v, *bufs_sems):
        xbufs = bufs_sems[:NXB]
        rbufs = bufs_sems[NXB:NXB + NRB]
        xsems = bufs_sems[NXB + NRB:2 * NXB + NRB]
        gsems = bufs_sems[2 * NXB + NRB:2 * NXB + 2 * NRB]
        osems = bufs_sems[2 * NXB + 2 * NRB:]

        wid = lax.axis_index("s") * info.num_cores + lax.axis_index("c")
        base = wid * rows_per_w
        pltpu.sync_copy(idx_hbm.at[pl.ds(base, rows_per_w)], idx_v)

        def gather_cpy(c, rb):
            return pltpu.make_async_copy(
                pe_hbm.at[idx_v.at[pl.ds(c * CHUNK, CHUNK)]],
                rbufs[rb], gsems[rb])

        def xload_cpy(c, xb):
            return pltpu.make_async_copy(
                x_hbm.at[pl.ds(base + c * CHUNK, CHUNK)], xbufs[xb],
                xsems[xb])

        def store_cpy(c, xb):
            return pltpu.make_async_copy(
                xbufs[xb], out_hbm.at[pl.ds(base + c * CHUNK, CHUNK)],
                osems[xb])

        gather_cpy(0, 0).start()
        xload_cpy(0, 0).start()

        def outer(i, carry):
            g = i * NXB
            for xb in range(NXB):
                cur = g + xb
                rb = xb % NRB
                nrb, nxb = (xb + 1) % NRB, (xb + 1) % NXB
                gather_cpy(cur, rb).wait()
                xload_cpy(cur, xb).wait()

                @pl.when(cur + 1 < n_chunks)
                def _():
                    gather_cpy(cur + 1, nrb).start()

                    @pl.when(cur + 1 >= NXB)
                    def _():
                        store_cpy(cur + 1 - NXB, nxb).wait()

                    xload_cpy(cur + 1, nxb).start()

                xbr, rbr = xbufs[xb], rbufs[rb]

                def row_body(r, c):
                    for j in range(D_MODEL // LANES):
                        sl = pl.ds(j * LANES, LANES)
                        plsc.addupdate(xbr.at[r, sl], rbr[r, sl])
                    return c

                lax.fori_loop(0, CHUNK, row_body, 0)
                store_cpy(cur, xb).start()
            return carry

        lax.fori_loop(0, n_chunks // NXB, outer, 0)
        for xb in range(NXB):
            store_cpy(n_chunks - NXB + xb, xb).wait()

    return k


def kernel(input_embeddings, position_ids, pe):
    b, s, d = input_embeddings.shape
    n = b * s
    x2d = input_embeddings.reshape(n, d)
    idx = position_ids.reshape(n).astype(jnp.int32)
    out = _pe_add_kernel(n)(x2d, idx, pe)
    return out.reshape(b, s, d)


# speedup vs baseline: 1.0249x; 1.0249x over previous
import functools

import jax
import jax.numpy as jnp
from jax import lax
from jax.experimental import pallas as pl
from jax.experimental.pallas import tpu as pltpu
from jax.experimental.pallas import tpu_sc as plsc

D_MODEL = 1024
LANES = 16
CHUNK = 16
NXB = 4
NRB = 2


def _pe_add_kernel(n_rows: int):
    info = plsc.get_sparse_core_info()
    num_workers = info.num_cores * info.num_subcores
    rows_per_w = n_rows // num_workers
    n_chunks = rows_per_w // CHUNK
    assert n_chunks % NXB == 0
    mesh = plsc.VectorSubcoreMesh(core_axis_name="c", subcore_axis_name="s")

    buf = lambda: pltpu.VMEM((CHUNK, D_MODEL), jnp.float32)

    @functools.partial(
        pl.kernel,
        mesh=mesh,
        out_type=jax.ShapeDtypeStruct((n_rows, D_MODEL), jnp.float32),
        scratch_types=(
            [pltpu.VMEM((rows_per_w,), jnp.int32)]
            + [buf() for _ in range(NXB + NRB)]
            + [pltpu.SemaphoreType.DMA for _ in range(NXB + NRB + NXB)]
        ),
    )
    def k(x_hbm, idx_hbm, pe_hbm, out_hbm, idx_v, *bufs_sems):
        xbufs = bufs_sems[:NXB]
        rbufs = bufs_sems[NXB:NXB + NRB]
        xsems = bufs_sems[NXB + NRB:2 * NXB + NRB]
        gsems = bufs_sems[2 * NXB + NRB:2 * NXB + 2 * NRB]
        osems = bufs_sems[2 * NXB + 2 * NRB:]

        wid = lax.axis_index("s") * info.num_cores + lax.axis_index("c")
        base = wid * rows_per_w
        pltpu.sync_copy(idx_hbm.at[pl.ds(base, rows_per_w)], idx_v)

        def gather_cpy(c, rb):
            return pltpu.make_async_copy(
                pe_hbm.at[idx_v.at[pl.ds(c * CHUNK, CHUNK)]],
                rbufs[rb], gsems[rb])

        def xload_cpy(c, xb):
            return pltpu.make_async_copy(
                x_hbm.at[pl.ds(base + c * CHUNK, CHUNK)], xbufs[xb],
                xsems[xb])

        def store_cpy(c, xb):
            return pltpu.make_async_copy(
                xbufs[xb], out_hbm.at[pl.ds(base + c * CHUNK, CHUNK)],
                osems[xb])

        gather_cpy(0, 0).start()
        xload_cpy(0, 0).start()

        def outer(i, carry):
            g = i * NXB
            for xb in range(NXB):
                cur = g + xb
                rb = xb % NRB
                nrb, nxb = (xb + 1) % NRB, (xb + 1) % NXB

                @pl.when(cur + 1 < n_chunks)
                def _():
                    gather_cpy(cur + 1, nrb).start()

                    @pl.when(cur + 1 >= NXB)
                    def _():
                        store_cpy(cur + 1 - NXB, nxb).wait()

                    xload_cpy(cur + 1, nxb).start()

                gather_cpy(cur, rb).wait()
                xload_cpy(cur, xb).wait()

                xbr, rbr = xbufs[xb], rbufs[rb]

                def row_body(r, c):
                    for j in range(D_MODEL // LANES):
                        sl = pl.ds(j * LANES, LANES)
                        plsc.addupdate(xbr.at[r, sl], rbr[r, sl])
                    return c

                lax.fori_loop(0, CHUNK, row_body, 0)
                store_cpy(cur, xb).start()
            return carry

        lax.fori_loop(0, n_chunks // NXB, outer, 0)
        for xb in range(NXB):
            store_cpy(n_chunks - NXB + xb, xb).wait()

    return k


def kernel(input_embeddings, position_ids, pe):
    b, s, d = input_embeddings.shape
    n = b * s
    x2d = input_embeddings.reshape(n, d)
    idx = position_ids.reshape(n).astype(jnp.int32)
    out = _pe_add_kernel(n)(x2d, idx, pe)
    return out.reshape(b, s, d)
